# Initial kernel scaffold; baseline (speedup 1.0000x reference)
import jax
import jax.numpy as jnp
from jax.experimental import pallas as pl


def kernel(hidden_states, expert_embeddings, W_dr, b_dr, temperature):
    raise NotImplementedError("write your pallas kernel here")


# speedup vs baseline: 1.5761x; 1.5761x over previous
import functools

import jax
import jax.numpy as jnp
from jax import lax
from jax.experimental import pallas as pl
from jax.experimental.pallas import tpu as pltpu

_NUM_EXPERTS = 16
_TOP_K = 2
_ALPHA = 1e-06
_BLOCK = 512


def _prologue_body(ee_ref, w_ref, b_ref, e_ref, wbf_ref):
    ee = ee_ref[...]
    nrm = jnp.sqrt(jnp.sum(ee * ee, axis=1, keepdims=True))
    ee_n = ee / jnp.maximum(nrm, 1e-12)
    w = w_ref[...]
    wbf = w.astype(jnp.bfloat16)
    e = lax.dot_general(ee_n.astype(jnp.bfloat16), wbf, (((1,), (1,)), ((), ())),
                        preferred_element_type=jnp.float32) + b_ref[...]
    e_ref[...] = e.astype(jnp.bfloat16)
    wbf_ref[...] = wbf


def _router_body(nsteps, hs_ref, w_ref, b_ref, e_ref, t_ref,
                 idx_ref, wt_ref, pi_ref, fi_ref, aux_ref, acc_ref):
    i = pl.program_id(0)

    @pl.when(i == 0)
    def _():
        acc_ref[...] = jnp.zeros_like(acc_ref)

    hs = hs_ref[...]
    sq = jnp.sum(hs * hs, axis=1, keepdims=True)
    inv = 1.0 / jnp.maximum(jnp.sqrt(sq), 1e-12)
    hs_n = (hs * inv).astype(jnp.bfloat16)
    x = lax.dot_general(hs_n, w_ref[...], (((1,), (1,)), ((), ())),
                        preferred_element_type=jnp.float32) + b_ref[...]
    logits = lax.dot_general(x.astype(jnp.bfloat16), e_ref[...],
                             (((1,), (1,)), ((), ())),
                             preferred_element_type=jnp.float32)
    logits = logits / t_ref[0]

    m = jnp.max(logits, axis=1, keepdims=True)
    ex = jnp.exp(logits - m)
    scores = ex / jnp.sum(ex, axis=1, keepdims=True)

    iota = lax.broadcasted_iota(jnp.int32, scores.shape, 1)
    m1 = jnp.max(scores, axis=1, keepdims=True)
    a1 = jnp.min(jnp.where(scores == m1, iota, _NUM_EXPERTS), axis=1, keepdims=True)
    masked = jnp.where(iota == a1, -1.0, scores)
    m2 = jnp.max(masked, axis=1, keepdims=True)
    a2 = jnp.min(jnp.where(masked == m2, iota, _NUM_EXPERTS), axis=1, keepdims=True)

    denom = m1 + m2 + 1e-06
    idx_ref[...] = jnp.concatenate([a1, a2], axis=1)
    wt_ref[...] = jnp.concatenate([m1 / denom, m2 / denom], axis=1)

    sel = (iota == a1).astype(jnp.float32) + (iota == a2).astype(jnp.float32)
    acc_ref[0:1, :] += jnp.sum(scores, axis=0, keepdims=True)
    acc_ref[1:2, :] += jnp.sum(sel, axis=0, keepdims=True)

    @pl.when(i == nsteps - 1)
    def _():
        total = jnp.float32(nsteps * idx_ref.shape[0])
        pi = acc_ref[0:1, :] / total
        ce = acc_ref[1:2, :] / (total * _TOP_K)
        fi = ce * _NUM_EXPERTS
        pi_ref[...] = pi
        fi_ref[...] = fi
        aux_ref[...] = jnp.sum(pi * fi, keepdims=True).reshape(1, 1) * _ALPHA


def kernel(hidden_states, expert_embeddings, W_dr, b_dr, temperature):
    bsz, seq_len, h = hidden_states.shape
    proj = W_dr.shape[0]
    tokens = bsz * seq_len
    hs = hidden_states.reshape(tokens, h)
    b2 = b_dr.reshape(1, proj)
    t1 = temperature.reshape(1)

    e_bf, w_bf = pl.pallas_call(
        _prologue_body,
        out_shape=(
            jax.ShapeDtypeStruct((_NUM_EXPERTS, proj), jnp.bfloat16),
            jax.ShapeDtypeStruct((proj, h), jnp.bfloat16),
        ),
        in_specs=[
            pl.BlockSpec(expert_embeddings.shape, lambda: (0, 0)),
            pl.BlockSpec(W_dr.shape, lambda: (0, 0)),
            pl.BlockSpec(b2.shape, lambda: (0, 0)),
        ],
        out_specs=(
            pl.BlockSpec((_NUM_EXPERTS, proj), lambda: (0, 0)),
            pl.BlockSpec((proj, h), lambda: (0, 0)),
        ),
    )(expert_embeddings, W_dr, b2)

    nsteps = tokens // _BLOCK
    idx, wt, pi, fi, aux = pl.pallas_call(
        functools.partial(_router_body, nsteps),
        grid=(nsteps,),
        in_specs=[
            pl.BlockSpec((_BLOCK, h), lambda i: (i, 0)),
            pl.BlockSpec((proj, h), lambda i: (0, 0)),
            pl.BlockSpec((1, proj), lambda i: (0, 0)),
            pl.BlockSpec((_NUM_EXPERTS, proj), lambda i: (0, 0)),
            pl.BlockSpec(memory_space=pltpu.SMEM),
        ],
        out_specs=(
            pl.BlockSpec((_BLOCK, _TOP_K), lambda i: (i, 0)),
            pl.BlockSpec((_BLOCK, _TOP_K), lambda i: (i, 0)),
            pl.BlockSpec((1, _NUM_EXPERTS), lambda i: (0, 0)),
            pl.BlockSpec((1, _NUM_EXPERTS), lambda i: (0, 0)),
            pl.BlockSpec((1, 1), lambda i: (0, 0)),
        ),
        out_shape=(
            jax.ShapeDtypeStruct((tokens, _TOP_K), jnp.int32),
            jax.ShapeDtypeStruct((tokens, _TOP_K), jnp.float32),
            jax.ShapeDtypeStruct((1, _NUM_EXPERTS), jnp.float32),
            jax.ShapeDtypeStruct((1, _NUM_EXPERTS), jnp.float32),
            jax.ShapeDtypeStruct((1, 1), jnp.float32),
        ),
        scratch_shapes=[pltpu.VMEM((2, _NUM_EXPERTS), jnp.float32)],
    )(hs, w_bf, b2, e_bf, t1)

    return (idx, wt, aux.reshape(()), fi.reshape(_NUM_EXPERTS), pi.reshape(_NUM_EXPERTS))


# speedup vs baseline: 1.5768x; 1.0005x over previous
import functools

import jax
import jax.numpy as jnp
from jax import lax
from jax.experimental import pallas as pl
from jax.experimental.pallas import tpu as pltpu

_NUM_EXPERTS = 16
_TOP_K = 2
_ALPHA = 1e-06
_BLOCK = 512


def _fused_body(nsteps, hs_ref, wbf_ref, b_ref, ee_ref, t_ref,
                idx_ref, wt_ref, pi_ref, fi_ref, aux_ref,
                ring_ref, e_ref, acc_ref):
    i = pl.program_id(0)

    @pl.when(i == 0)
    def _():
        ee = ee_ref[...]
        nrm = jnp.sqrt(jnp.sum(ee * ee, axis=1, keepdims=True))
        ee_n = ee / jnp.maximum(nrm, 1e-12)
        e = lax.dot_general(ee_n.astype(jnp.bfloat16), wbf_ref[...],
                            (((1,), (1,)), ((), ())),
                            preferred_element_type=jnp.float32) + b_ref[...]
        e_ref[...] = e.astype(jnp.bfloat16)
        acc_ref[...] = jnp.zeros_like(acc_ref)

    hs = hs_ref[...]
    sq = jnp.sum(hs * hs, axis=1, keepdims=True)
    inv = 1.0 / jnp.maximum(jnp.sqrt(sq), 1e-12)
    ring_ref[i % 2] = (hs * inv).astype(jnp.bfloat16)

    @pl.when(i > 0)
    def _():
        hs_n = ring_ref[(i - 1) % 2]
        x = lax.dot_general(hs_n, wbf_ref[...], (((1,), (1,)), ((), ())),
                            preferred_element_type=jnp.float32) + b_ref[...]
        lt = lax.dot_general(e_ref[...], x.astype(jnp.bfloat16),
                             (((1,), (1,)), ((), ())),
                             preferred_element_type=jnp.float32)
        lt = lt / t_ref[0]

        m = jnp.max(lt, axis=0, keepdims=True)
        ex = jnp.exp(lt - m)
        scores = ex / jnp.sum(ex, axis=0, keepdims=True)

        iota = lax.broadcasted_iota(jnp.int32, scores.shape, 0)
        m1 = jnp.max(scores, axis=0, keepdims=True)
        a1 = jnp.min(jnp.where(scores == m1, iota, _NUM_EXPERTS),
                     axis=0, keepdims=True)
        masked = jnp.where(iota == a1, -1.0, scores)
        m2 = jnp.max(masked, axis=0, keepdims=True)
        a2 = jnp.min(jnp.where(masked == m2, iota, _NUM_EXPERTS),
                     axis=0, keepdims=True)

        denom = m1 + m2 + 1e-06
        idx_t = jnp.concatenate([a1, a2], axis=0).astype(jnp.float32)
        idx_ref[...] = lax.transpose(idx_t, (1, 0)).astype(jnp.int32)
        wt_t = jnp.concatenate([m1 / denom, m2 / denom], axis=0)
        wt_ref[...] = lax.transpose(wt_t, (1, 0))

        sel = (iota == a1).astype(jnp.float32) + (iota == a2).astype(jnp.float32)
        acc_ref[0] += scores
        acc_ref[1] += sel

    @pl.when(i == nsteps)
    def _():
        total = jnp.float32(nsteps * _BLOCK)
        pi = jnp.sum(acc_ref[0], axis=1, keepdims=True) / total
        ce = jnp.sum(acc_ref[1], axis=1, keepdims=True) / (total * _TOP_K)
        fi = ce * _NUM_EXPERTS
        pi_ref[...] = pi
        fi_ref[...] = fi
        aux_ref[...] = jnp.sum(pi * fi, keepdims=True).reshape(1, 1) * _ALPHA


def kernel(hidden_states, expert_embeddings, W_dr, b_dr, temperature):
    bsz, seq_len, h = hidden_states.shape
    proj = W_dr.shape[0]
    tokens = bsz * seq_len
    hs = hidden_states.reshape(tokens, h)
    w_bf = W_dr.astype(jnp.bfloat16)
    b2 = b_dr.reshape(1, proj)
    t1 = temperature.reshape(1)

    nsteps = tokens // _BLOCK
    idx, wt, pi, fi, aux = pl.pallas_call(
        functools.partial(_fused_body, nsteps),
        grid=(nsteps + 1,),
        in_specs=[
            pl.BlockSpec((_BLOCK, h), lambda i: (jnp.minimum(i, nsteps - 1), 0)),
            pl.BlockSpec((proj, h), lambda i: (0, 0)),
            pl.BlockSpec((1, proj), lambda i: (0, 0)),
            pl.BlockSpec((_NUM_EXPERTS, h), lambda i: (0, 0)),
            pl.BlockSpec(memory_space=pltpu.SMEM),
        ],
        out_specs=(
            pl.BlockSpec((_BLOCK, _TOP_K), lambda i: (jnp.maximum(i - 1, 0), 0)),
            pl.BlockSpec((_BLOCK, _TOP_K), lambda i: (jnp.maximum(i - 1, 0), 0)),
            pl.BlockSpec((_NUM_EXPERTS, 1), lambda i: (0, 0)),
            pl.BlockSpec((_NUM_EXPERTS, 1), lambda i: (0, 0)),
            pl.BlockSpec((1, 1), lambda i: (0, 0)),
        ),
        out_shape=(
            jax.ShapeDtypeStruct((tokens, _TOP_K), jnp.int32),
            jax.ShapeDtypeStruct((tokens, _TOP_K), jnp.float32),
            jax.ShapeDtypeStruct((_NUM_EXPERTS, 1), jnp.float32),
            jax.ShapeDtypeStruct((_NUM_EXPERTS, 1), jnp.float32),
            jax.ShapeDtypeStruct((1, 1), jnp.float32),
        ),
        scratch_shapes=[
            pltpu.VMEM((2, _BLOCK, h), jnp.bfloat16),
            pltpu.VMEM((_NUM_EXPERTS, proj), jnp.bfloat16),
            pltpu.VMEM((2, _NUM_EXPERTS, _BLOCK), jnp.float32),
        ],
    )(hs, w_bf, b2, expert_embeddings, t1)

    return (idx, wt, aux.reshape(()), fi.reshape(_NUM_EXPERTS), pi.reshape(_NUM_EXPERTS))
